# baseline (speedup 1.0000x reference)
import jax
import jax.numpy as jnp
from jax.experimental import pallas as pl

_HEIGHT = 4096
_WIDTH = 4096
_BATCH = 2048
_NBLK = 512
_NSUB = 256


def _decode_matmul_kernel(x_ref, code_ref, mm_ref, out_ref):
    for nb in range(0, _NBLK, _NSUB):
        code = code_ref[nb:nb + _NSUB, :]
        mins = mm_ref[nb:nb + _NSUB, 0:1]
        maxs = mm_ref[nb:nb + _NSUB, 1:2]
        w = (mins * (code == 1).astype(jnp.bfloat16)
             + maxs * (code == 2).astype(jnp.bfloat16))
        out_ref[:, nb:nb + _NSUB] = jax.lax.dot_general(
            x_ref[...],
            w,
            (((1,), (1,)), ((), ())),
            preferred_element_type=jnp.float32,
        ).astype(jnp.bfloat16)


def kernel(x, w_tern, ter_minmax):
    mm = ter_minmax.reshape(_HEIGHT, 2)
    nj = _HEIGHT // _NBLK
    return pl.pallas_call(
        _decode_matmul_kernel,
        grid=(nj,),
        in_specs=[
            pl.BlockSpec((_BATCH, _WIDTH), lambda j: (0, 0)),
            pl.BlockSpec((_NBLK, _WIDTH), lambda j: (j, 0)),
            pl.BlockSpec((_NBLK, 2), lambda j: (j, 0)),
        ],
        out_specs=pl.BlockSpec((_BATCH, _NBLK), lambda j: (0, j)),
        out_shape=jax.ShapeDtypeStruct((_BATCH, _HEIGHT), jnp.bfloat16),
    )(x, w_tern, mm)


# speedup vs baseline: 1.0349x; 1.0349x over previous
import jax
import jax.numpy as jnp
from jax.experimental import pallas as pl

_HEIGHT = 4096
_WIDTH = 4096
_BATCH = 2048
_NBLK = 512
_MBLK = 1024


def _decode_matmul_kernel(x_ref, code_ref, mm_ref, out_ref):
    c = code_ref[...].astype(jnp.bfloat16)
    mins_b = jnp.broadcast_to(mm_ref[:, 0:1], (_NBLK, _WIDTH))
    maxs_b = jnp.broadcast_to(mm_ref[:, 1:2], (_NBLK, _WIDTH))
    zeros = jnp.zeros((_NBLK, _WIDTH), jnp.bfloat16)
    w = jnp.where(c == 1.0, mins_b, jnp.where(c == 2.0, maxs_b, zeros))
    for mb in range(0, _BATCH, _MBLK):
        out_ref[mb:mb + _MBLK, :] = jax.lax.dot_general(
            x_ref[mb:mb + _MBLK, :],
            w,
            (((1,), (1,)), ((), ())),
            preferred_element_type=jnp.float32,
        ).astype(jnp.bfloat16)


def kernel(x, w_tern, ter_minmax):
    mm = ter_minmax.reshape(_HEIGHT, 2)
    nj = _HEIGHT // _NBLK
    return pl.pallas_call(
        _decode_matmul_kernel,
        grid=(nj,),
        in_specs=[
            pl.BlockSpec((_BATCH, _WIDTH), lambda j: (0, 0)),
            pl.BlockSpec((_NBLK, _WIDTH), lambda j: (j, 0)),
            pl.BlockSpec((_NBLK, 2), lambda j: (j, 0)),
        ],
        out_specs=pl.BlockSpec((_BATCH, _NBLK), lambda j: (0, j)),
        out_shape=jax.ShapeDtypeStruct((_BATCH, _HEIGHT), jnp.bfloat16),
    )(x, w_tern, mm)


# speedup vs baseline: 1.0415x; 1.0064x over previous
import jax
import jax.numpy as jnp
from jax.experimental import pallas as pl

_HEIGHT = 4096
_WIDTH = 4096
_BATCH = 2048
_NBLK = 512
_NSUB = 256
_MBLK = 1024


def _decode_matmul_kernel(x_ref, code_ref, mm_ref, out_ref):
    zeros = jnp.zeros((_NSUB, _WIDTH), jnp.bfloat16)
    for nb in range(0, _NBLK, _NSUB):
        c = code_ref[nb:nb + _NSUB, :].astype(jnp.bfloat16)
        mins_b = jnp.broadcast_to(mm_ref[nb:nb + _NSUB, 0:1], (_NSUB, _WIDTH))
        maxs_b = jnp.broadcast_to(mm_ref[nb:nb + _NSUB, 1:2], (_NSUB, _WIDTH))
        w = jnp.where(c == 1.0, mins_b, jnp.where(c == 2.0, maxs_b, zeros))
        for mb in range(0, _BATCH, _MBLK):
            out_ref[mb:mb + _MBLK, nb:nb + _NSUB] = jax.lax.dot_general(
                x_ref[mb:mb + _MBLK, :],
                w,
                (((1,), (1,)), ((), ())),
                preferred_element_type=jnp.float32,
            ).astype(jnp.bfloat16)


def kernel(x, w_tern, ter_minmax):
    mm = ter_minmax.reshape(_HEIGHT, 2)
    nj = _HEIGHT // _NBLK
    return pl.pallas_call(
        _decode_matmul_kernel,
        grid=(nj,),
        in_specs=[
            pl.BlockSpec((_BATCH, _WIDTH), lambda j: (0, 0)),
            pl.BlockSpec((_NBLK, _WIDTH), lambda j: (j, 0)),
            pl.BlockSpec((_NBLK, 2), lambda j: (j, 0)),
        ],
        out_specs=pl.BlockSpec((_BATCH, _NBLK), lambda j: (0, j)),
        out_shape=jax.ShapeDtypeStruct((_BATCH, _HEIGHT), jnp.bfloat16),
    )(x, w_tern, mm)
